# baseline (speedup 1.0000x reference)
import functools

import jax
import jax.numpy as jnp
from jax import lax
from jax.experimental import pallas as pl
from jax.experimental.pallas import tpu as pltpu
from jax.experimental.pallas import tpu_sc as plsc

_N = 10000
_E = 320000
_NC = 2
_NS = 16
_NW = _NC * _NS
_L = 16
_N2 = 2 * _N

_DCH = 19968
_DBUF = _E - (_NS - 1) * _DCH
_MCH = _DCH // 2
_MBUF = _DBUF // 2

_NH = 10240
_STR = _NH // _NS


def _sc_mesh():
    return plsc.VectorSubcoreMesh(
        core_axis_name="c", subcore_axis_name="s",
        num_cores=_NC, num_subcores=_NS)


def _zero_vmem(ref, n):
    zv = jnp.zeros((_L,), jnp.float32)

    @plsc.parallel_loop(0, n // _L, unroll=16)
    def _(i):
        ref[pl.ds(i * _L, _L)] = zv


def _newton_rsqrt(x):
    i = plsc.bitcast(x, jnp.int32)
    y = plsc.bitcast(jnp.int32(0x5F3759DF) - (i >> 1), jnp.float32)
    for _ in range(3):
        y = y * (1.5 - 0.5 * x * y * y)
    return y


def _sc_body(el_hbm, w_hbm, h2_hbm, b1_hbm, out_hbm,
             rc_v, w_v, dd_v, tmp2_v, g0_v, g1_v, acc_v, hs_v, b1_v,
             ds_v, g0s_v, g1s_v,
             spm_deg, spm_dis, spm_g0, spm_g1, sem):
    cid = lax.axis_index("c")
    sid = lax.axis_index("s")
    wid = sid * _NC + cid

    dbase = sid * _DCH
    mlen = jnp.where(sid == _NS - 1, _MBUF, _MCH)
    nb = sid * _STR

    hbase_a = pl.multiple_of(dbase + (1 - cid) * mlen, 128)
    hbase_b = pl.multiple_of(dbase + cid * mlen, 128)

    cps = [
        pltpu.async_copy(el_hbm.at[:, pl.ds(hbase_a, _MBUF)], rc_v, sem),
        pltpu.async_copy(w_hbm.at[pl.ds(hbase_a, _MBUF)], w_v, sem),
        pltpu.async_copy(h2_hbm.at[:, pl.ds(nb, _STR)], hs_v, sem),
        pltpu.async_copy(b1_hbm.at[0], b1_v, sem),
    ]
    _zero_vmem(dd_v, _NH)
    _zero_vmem(acc_v, _N2)
    for cp in cps:
        cp.wait()

    iota = lax.iota(jnp.int32, _L)
    zeros16 = jnp.zeros((_L,), jnp.int32)
    ones16 = jnp.ones((_L,), jnp.int32)

    def _deg_loop():
        @plsc.parallel_loop(0, _MBUF // _L, unroll=16)
        def _(i):
            o = i * _L
            mask = (o + iota) < mlen
            c = rc_v[1, pl.ds(o, _L)]
            ww = w_v[pl.ds(o, _L)]
            plsc.addupdate_scatter(dd_v, [c], ww, mask=mask)

    _deg_loop()
    cp1 = pltpu.async_copy(el_hbm.at[:, pl.ds(hbase_b, _MBUF)], rc_v, sem)
    cp2 = pltpu.async_copy(w_hbm.at[pl.ds(hbase_b, _MBUF)], w_v, sem)
    cp1.wait()
    cp2.wait()
    _deg_loop()

    pltpu.sync_copy(dd_v, spm_deg.at[sid])
    plsc.subcore_barrier()
    pltpu.sync_copy(spm_deg.at[:, pl.ds(nb, _STR)], tmp2_v)

    @plsc.parallel_loop(0, _STR // _L, unroll=4)
    def _(j):
        o = j * _L
        deg = tmp2_v[0, pl.ds(o, _L)]
        for k in range(1, _NS):
            deg = deg + tmp2_v[k, pl.ds(o, _L)]
        dis = _newton_rsqrt(deg + 1.0)
        ds_v[pl.ds(o, _L)] = dis
        g0s_v[pl.ds(o, _L)] = dis * hs_v[0, pl.ds(o, _L)]
        g1s_v[pl.ds(o, _L)] = dis * hs_v[1, pl.ds(o, _L)]

    pltpu.sync_copy(ds_v, spm_dis.at[pl.ds(nb, _STR)])
    pltpu.sync_copy(g0s_v, spm_g0.at[pl.ds(nb, _STR)])
    pltpu.sync_copy(g1s_v, spm_g1.at[pl.ds(nb, _STR)])
    plsc.subcore_barrier()
    pltpu.sync_copy(spm_dis, dd_v)
    pltpu.sync_copy(spm_g0, g0_v)
    pltpu.sync_copy(spm_g1, g1_v)

    @pl.when(cid == 0)
    def _():
        b0 = plsc.load_gather(b1_v, [zeros16])
        b1b = plsc.load_gather(b1_v, [ones16])

        @plsc.parallel_loop(0, _STR // _L, unroll=4)
        def _(j):
            jj = j * _L
            n16 = nb + jj + iota
            mask = n16 < _N
            n2 = n16 * 2
            s0 = dd_v[pl.ds(nb + jj, _L)] * g0_v[pl.ds(nb + jj, _L)] + b0
            s1 = dd_v[pl.ds(nb + jj, _L)] * g1_v[pl.ds(nb + jj, _L)] + b1b
            plsc.store_scatter(acc_v, [n2], s0, mask=mask)
            plsc.store_scatter(acc_v, [n2 + 1], s1, mask=mask)

    @plsc.parallel_loop(0, _MBUF // _L, unroll=16)
    def _(i):
        o = i * _L
        mask = (o + iota) < mlen
        r = rc_v[0, pl.ds(o, _L)]
        c = rc_v[1, pl.ds(o, _L)]
        ww = w_v[pl.ds(o, _L)]
        wd = ww * plsc.load_gather(dd_v, [c])
        m0 = wd * plsc.load_gather(g0_v, [r])
        m1 = wd * plsc.load_gather(g1_v, [r])
        c2 = c * 2
        plsc.addupdate_scatter(acc_v, [c2], m0, mask=mask)
        plsc.addupdate_scatter(acc_v, [c2 + 1], m1, mask=mask)

    pltpu.sync_copy(acc_v, out_hbm.at[wid])


_sc_call = functools.partial(
    pl.kernel,
    out_type=jax.ShapeDtypeStruct((_NW, _N2), jnp.float32),
    mesh=_sc_mesh(),
    compiler_params=pltpu.CompilerParams(needs_layout_passes=False),
    scratch_types=[
        pltpu.VMEM((2, _MBUF), jnp.int32),
        pltpu.VMEM((_MBUF,), jnp.float32),
        pltpu.VMEM((_NH,), jnp.float32),
        pltpu.VMEM((_NS, _STR), jnp.float32),
        pltpu.VMEM((_NH,), jnp.float32),
        pltpu.VMEM((_NH,), jnp.float32),
        pltpu.VMEM((_N2,), jnp.float32),
        pltpu.VMEM((2, _STR), jnp.float32),
        pltpu.VMEM((_L,), jnp.float32),
        pltpu.VMEM((_STR,), jnp.float32),
        pltpu.VMEM((_STR,), jnp.float32),
        pltpu.VMEM((_STR,), jnp.float32),
        pltpu.VMEM_SHARED((_NS, _NH), jnp.float32),
        pltpu.VMEM_SHARED((_NH,), jnp.float32),
        pltpu.VMEM_SHARED((_NH,), jnp.float32),
        pltpu.VMEM_SHARED((_NH,), jnp.float32),
        pltpu.SemaphoreType.DMA,
    ],
)(_sc_body)


def _mm_body(x_ref, w1_ref, h_ref):
    h = lax.dot_general(
        w1_ref[...], x_ref[...], (((0,), (1,)), ((), ())),
        preferred_element_type=jnp.float32)
    h_ref[...] = jnp.pad(h, ((0, 0), (0, _NH - _N)))


def _mm_call(x, w1):
    return pl.pallas_call(
        _mm_body,
        out_shape=jax.ShapeDtypeStruct((2, _NH), jnp.float32),
    )(x, w1)


def _final_body(ap_ref, wfc_ref, bfc_ref, y_ref):
    out = jnp.maximum(jnp.sum(ap_ref[...], axis=0, keepdims=True), 0.0)
    s = jnp.sum(out * wfc_ref[...], keepdims=True).reshape(1, 1)
    y_ref[...] = jax.nn.sigmoid(s + bfc_ref[...])


def _final_call(ap, wfc, bfc):
    return pl.pallas_call(
        _final_body,
        out_shape=jax.ShapeDtypeStruct((1, 1), jnp.float32),
    )(ap, wfc, bfc)


def kernel(x, edge_list, edge_attr, W1, b1, W_fc, b_fc):
    h2 = _mm_call(x, W1)
    b1p = jnp.pad(b1, (0, _L - 2)).reshape(1, _L)
    ap = _sc_call(edge_list, edge_attr, h2, b1p)
    y = _final_call(ap, W_fc, b_fc.reshape(1, 1))
    return y[0, 0]
